# baseline (speedup 1.0000x reference)
import functools

import jax
import jax.numpy as jnp
from jax import lax
from jax.experimental import pallas as pl
from jax.experimental.pallas import tpu as pltpu
from jax.experimental.pallas import tpu_sc as plsc

_B = 16384
_NC = 2
_NS = 16
_NW = _NC * _NS
_BPW = _B // _NW
_L = 16

_VALUE_RANGE = 8.0
_A_RANGE = 3.0
_D = 1.702


def _sigmoid(x):
    return 1.0 / (1.0 + jnp.exp(-x))


def _merge_tables(theta_w, a_w, b_w, c_w):
    cat = jnp.concatenate([theta_w, a_w, b_w, c_w], axis=0)
    pad = (-cat.shape[0]) % 1024
    return jnp.pad(cat, ((0, pad), (0, 0))).reshape(-1)


_OFF_A = 1000000
_OFF_B = 1100000
_OFF_C = 1200000


def _body(user_hbm, item_hbm, pair_hbm, tab_hbm,
          irf_hbm, th_hbm, tp_hbm,
          uidx, iidx, pidx, bidx, cidx, th_v, tp_v, a_v, b_v, c_v,
          irf_o, th_o, tp_o, sem, sem2):
    wid = lax.axis_index("s") * _NC + lax.axis_index("c")
    base = wid * _BPW

    icps = (pltpu.async_copy(user_hbm.at[pl.ds(base, _BPW)], uidx, sem),
            pltpu.async_copy(item_hbm.at[pl.ds(base, _BPW)], iidx, sem),
            pltpu.async_copy(pair_hbm.at[pl.ds(base, _BPW)], pidx, sem))
    for cp in icps:
        cp.wait()

    def shift(i):
        sl = pl.ds(i * _L, _L)
        v = iidx[sl]
        iidx[sl] = v + _OFF_A
        bidx[sl] = v + _OFF_B
        cidx[sl] = v + _OFF_C

    plsc.parallel_loop(0, _BPW // _L, 1, unroll=4)(shift)

    tcps = (pltpu.async_copy(tab_hbm.at[uidx], th_v, sem),
            pltpu.async_copy(tab_hbm.at[pidx], tp_v, sem))
    icps2 = (pltpu.async_copy(tab_hbm.at[iidx], a_v, sem2),
             pltpu.async_copy(tab_hbm.at[bidx], b_v, sem2),
             pltpu.async_copy(tab_hbm.at[cidx], c_v, sem2))
    for cp in tcps:
        cp.wait()

    def chunk_theta(i):
        sl = pl.ds(i * _L, _L)
        theta = _VALUE_RANGE * (_sigmoid(th_v[sl]) - 0.5)
        theta_pair = _VALUE_RANGE * (_sigmoid(tp_v[sl]) - 0.5)
        th_v[sl] = theta
        th_o[sl] = _sigmoid(theta)
        tp_o[sl] = _sigmoid(theta_pair)

    plsc.parallel_loop(0, _BPW // _L, 1, unroll=4)(chunk_theta)

    for cp in icps2:
        cp.wait()

    def chunk_irf(i):
        sl = pl.ds(i * _L, _L)
        theta = th_v[sl]
        a = _A_RANGE * _sigmoid(a_v[sl])
        b = _VALUE_RANGE * (_sigmoid(b_v[sl]) - 0.5)
        c = _sigmoid(c_v[sl])
        irf_o[sl] = c + (1.0 - c) / (1.0 + jnp.exp(-_D * a * (theta - b)))

    plsc.parallel_loop(0, _BPW // _L, 1, unroll=4)(chunk_irf)

    ocps = (pltpu.async_copy(irf_o, irf_hbm.at[pl.ds(base, _BPW)], sem),
            pltpu.async_copy(th_o, th_hbm.at[pl.ds(base, _BPW)], sem),
            pltpu.async_copy(tp_o, tp_hbm.at[pl.ds(base, _BPW)], sem))
    for cp in ocps:
        cp.wait()


_irt_sc = functools.partial(
    pl.kernel,
    mesh=plsc.VectorSubcoreMesh(core_axis_name="c", subcore_axis_name="s"),
    out_type=(jax.ShapeDtypeStruct((_B,), jnp.float32),
              jax.ShapeDtypeStruct((_B,), jnp.float32),
              jax.ShapeDtypeStruct((_B,), jnp.float32)),
    scratch_types=[
        pltpu.VMEM((_BPW,), jnp.int32),
        pltpu.VMEM((_BPW,), jnp.int32),
        pltpu.VMEM((_BPW,), jnp.int32),
        pltpu.VMEM((_BPW,), jnp.int32),
        pltpu.VMEM((_BPW,), jnp.int32),
        pltpu.VMEM((_BPW,), jnp.float32),
        pltpu.VMEM((_BPW,), jnp.float32),
        pltpu.VMEM((_BPW,), jnp.float32),
        pltpu.VMEM((_BPW,), jnp.float32),
        pltpu.VMEM((_BPW,), jnp.float32),
        pltpu.VMEM((_BPW,), jnp.float32),
        pltpu.VMEM((_BPW,), jnp.float32),
        pltpu.VMEM((_BPW,), jnp.float32),
        pltpu.SemaphoreType.DMA,
        pltpu.SemaphoreType.DMA,
    ],
)(_body)


def kernel(user, item, user_pair, theta_w, a_w, b_w, c_w):
    return _irt_sc(user, item, user_pair,
                   _merge_tables(theta_w, a_w, b_w, c_w))


# speedup vs baseline: 3.0703x; 3.0703x over previous
import functools

import jax
import jax.numpy as jnp
from jax import lax
from jax.experimental import pallas as pl
from jax.experimental.pallas import tpu as pltpu
from jax.experimental.pallas import tpu_sc as plsc

_B = 16384
_NC = 2
_NS = 16
_NW = _NC * _NS
_BPW = _B // _NW
_L = 16

_VALUE_RANGE = 8.0
_A_RANGE = 3.0
_D = 1.702


def _sigmoid(x):
    return 1.0 / (1.0 + jnp.exp(-x))


def _flat_pad(t):
    n = t.shape[0]
    pad = (-n) % 1024
    return jnp.pad(t, ((0, pad), (0, 0))).reshape(-1)


def _body(user_hbm, item_hbm, pair_hbm, theta_hbm, a_hbm, b_hbm, c_hbm,
          irf_hbm, th_hbm, tp_hbm,
          uidx, iidx, pidx, th_v, tp_v, a_v, b_v, c_v,
          irf_o, th_o, tp_o, sem, sem2):
    wid = lax.axis_index("s") * _NC + lax.axis_index("c")
    base = wid * _BPW

    icps = (pltpu.async_copy(user_hbm.at[pl.ds(base, _BPW)], uidx, sem),
            pltpu.async_copy(item_hbm.at[pl.ds(base, _BPW)], iidx, sem),
            pltpu.async_copy(pair_hbm.at[pl.ds(base, _BPW)], pidx, sem))
    for cp in icps:
        cp.wait()

    tcps = (pltpu.async_copy(theta_hbm.at[uidx], th_v, sem),
            pltpu.async_copy(theta_hbm.at[pidx], tp_v, sem))
    icps2 = (pltpu.async_copy(a_hbm.at[iidx], a_v, sem2),
             pltpu.async_copy(b_hbm.at[iidx], b_v, sem2),
             pltpu.async_copy(c_hbm.at[iidx], c_v, sem2))
    for cp in tcps:
        cp.wait()

    def chunk_theta(i):
        sl = pl.ds(i * _L, _L)
        theta = _VALUE_RANGE * (_sigmoid(th_v[sl]) - 0.5)
        theta_pair = _VALUE_RANGE * (_sigmoid(tp_v[sl]) - 0.5)
        th_v[sl] = theta
        th_o[sl] = _sigmoid(theta)
        tp_o[sl] = _sigmoid(theta_pair)

    plsc.parallel_loop(0, _BPW // _L, 1, unroll=4)(chunk_theta)

    for cp in icps2:
        cp.wait()

    def chunk_irf(i):
        sl = pl.ds(i * _L, _L)
        theta = th_v[sl]
        a = _A_RANGE * _sigmoid(a_v[sl])
        b = _VALUE_RANGE * (_sigmoid(b_v[sl]) - 0.5)
        c = _sigmoid(c_v[sl])
        irf_o[sl] = c + (1.0 - c) / (1.0 + jnp.exp(-_D * a * (theta - b)))

    plsc.parallel_loop(0, _BPW // _L, 1, unroll=4)(chunk_irf)

    ocps = (pltpu.async_copy(irf_o, irf_hbm.at[pl.ds(base, _BPW)], sem),
            pltpu.async_copy(th_o, th_hbm.at[pl.ds(base, _BPW)], sem),
            pltpu.async_copy(tp_o, tp_hbm.at[pl.ds(base, _BPW)], sem))
    for cp in ocps:
        cp.wait()


_irt_sc = functools.partial(
    pl.kernel,
    mesh=plsc.VectorSubcoreMesh(core_axis_name="c", subcore_axis_name="s"),
    out_type=(jax.ShapeDtypeStruct((_B,), jnp.float32),
              jax.ShapeDtypeStruct((_B,), jnp.float32),
              jax.ShapeDtypeStruct((_B,), jnp.float32)),
    scratch_types=[
        pltpu.VMEM((_BPW,), jnp.int32),
        pltpu.VMEM((_BPW,), jnp.int32),
        pltpu.VMEM((_BPW,), jnp.int32),
        pltpu.VMEM((_BPW,), jnp.float32),
        pltpu.VMEM((_BPW,), jnp.float32),
        pltpu.VMEM((_BPW,), jnp.float32),
        pltpu.VMEM((_BPW,), jnp.float32),
        pltpu.VMEM((_BPW,), jnp.float32),
        pltpu.VMEM((_BPW,), jnp.float32),
        pltpu.VMEM((_BPW,), jnp.float32),
        pltpu.VMEM((_BPW,), jnp.float32),
        pltpu.SemaphoreType.DMA,
        pltpu.SemaphoreType.DMA,
    ],
)(_body)


def kernel(user, item, user_pair, theta_w, a_w, b_w, c_w):
    return _irt_sc(user, item, user_pair,
                   _flat_pad(theta_w), _flat_pad(a_w),
                   _flat_pad(b_w), _flat_pad(c_w))
